# baseline (speedup 1.0000x reference)
import jax
import jax.numpy as jnp
from jax import lax
from jax.experimental import pallas as pl
from jax.experimental.pallas import tpu as pltpu

_T, _D, _F, _E = 1024, 768, 1024, 8
_LIMIT = 7.0
_ALPHA = 1.702
_FH = _F // 2


def _router_body(x_ref, w_ref, b_ref, comb_ref):
    x = x_ref[...]
    logits = jnp.dot(x, w_ref[...], preferred_element_type=jnp.float32) + b_ref[...]
    idx = lax.broadcasted_iota(jnp.int32, (_T, _E), 1)
    m1 = jnp.max(logits, axis=1, keepdims=True)
    a1 = jnp.min(jnp.where(logits == m1, idx, _E), axis=1, keepdims=True)
    l2 = jnp.where(idx == a1, -jnp.inf, logits)
    m2 = jnp.max(l2, axis=1, keepdims=True)
    a2 = jnp.min(jnp.where(l2 == m2, idx, _E), axis=1, keepdims=True)
    w1 = jax.nn.sigmoid(m1 - m2)
    comb_ref[...] = jnp.where(idx == a1, w1, 0.0) + jnp.where(idx == a2, 1.0 - w1, 0.0)


def _expert_body(comb_ref, x_ref, w1g_ref, w1u_ref, b1_ref, w2a_ref, w2b_ref, b2_ref, out_ref):
    e = pl.program_id(0)
    x = x_ref[...]
    gate = jnp.dot(x, w1g_ref[0], preferred_element_type=jnp.float32) + b1_ref[0, :, :_F]
    up = jnp.dot(x, w1u_ref[0], preferred_element_type=jnp.float32) + b1_ref[0, :, _F:]
    gate = jnp.minimum(gate, _LIMIT)
    up = jnp.clip(up, -_LIMIT, _LIMIT)
    act = (up + 1.0) * (gate * jax.nn.sigmoid(_ALPHA * gate))
    out = (jnp.dot(act[:, :_FH], w2a_ref[0], preferred_element_type=jnp.float32)
           + jnp.dot(act[:, _FH:], w2b_ref[0], preferred_element_type=jnp.float32)
           + b2_ref[0])
    idx = lax.broadcasted_iota(jnp.int32, (_T, _E), 1)
    c = jnp.sum(jnp.where(idx == e, comb_ref[...], 0.0), axis=1, keepdims=True)
    contrib = c * out

    @pl.when(e == 0)
    def _():
        out_ref[...] = contrib

    @pl.when(e > 0)
    def _():
        out_ref[...] += contrib


@jax.jit
def kernel(x_TD, kernel_DE, bias_E, mlp1_weight_EDF2, mlp1_bias_EF2, mlp2_weight_EFD, mlp2_bias_ED):
    x = x_TD.astype(jnp.float32)
    comb = pl.pallas_call(
        _router_body,
        out_shape=jax.ShapeDtypeStruct((_T, _E), jnp.float32),
    )(x, kernel_DE, bias_E.reshape(1, _E))

    w1 = mlp1_weight_EDF2
    out = pl.pallas_call(
        _expert_body,
        grid=(_E,),
        in_specs=[
            pl.BlockSpec((_T, _E), lambda e: (0, 0)),
            pl.BlockSpec((_T, _D), lambda e: (0, 0)),
            pl.BlockSpec((1, _D, _F), lambda e: (e, 0, 0)),
            pl.BlockSpec((1, _D, _F), lambda e: (e, 0, 1)),
            pl.BlockSpec((1, 1, 2 * _F), lambda e: (e, 0, 0)),
            pl.BlockSpec((1, _FH, _D), lambda e: (e, 0, 0)),
            pl.BlockSpec((1, _FH, _D), lambda e: (e, 1, 0)),
            pl.BlockSpec((1, 1, _D), lambda e: (e, 0, 0)),
        ],
        out_specs=pl.BlockSpec((_T, _D), lambda e: (0, 0)),
        out_shape=jax.ShapeDtypeStruct((_T, _D), jnp.float32),
        compiler_params=pltpu.CompilerParams(
            dimension_semantics=("arbitrary",),
        ),
    )(comb, x, w1, w1, mlp1_bias_EF2.reshape(_E, 1, 2 * _F),
      mlp2_weight_EFD, mlp2_weight_EFD, mlp2_bias_ED.reshape(_E, 1, _D))
    return out.astype(jnp.float32)


# speedup vs baseline: 1.0681x; 1.0681x over previous
import jax
import jax.numpy as jnp
from jax import lax
from jax.experimental import pallas as pl
from jax.experimental.pallas import tpu as pltpu

_T, _D, _F, _E = 1024, 768, 1024, 8
_LIMIT = 7.0
_ALPHA = 1.702


def _moe_body(x_ref, rw_ref, rb_ref, w1_ref, b1_ref, w2_ref, b2_ref, out_ref, comb_ref):
    e = pl.program_id(0)
    x = x_ref[...]
    idx = lax.broadcasted_iota(jnp.int32, (_T, _E), 1)

    @pl.when(e == 0)
    def _():
        logits = jnp.dot(x, rw_ref[...], preferred_element_type=jnp.float32) + rb_ref[...]
        m1 = jnp.max(logits, axis=1, keepdims=True)
        a1 = jnp.min(jnp.where(logits == m1, idx, _E), axis=1, keepdims=True)
        l2 = jnp.where(idx == a1, -jnp.inf, logits)
        m2 = jnp.max(l2, axis=1, keepdims=True)
        a2 = jnp.min(jnp.where(l2 == m2, idx, _E), axis=1, keepdims=True)
        w1 = jax.nn.sigmoid(m1 - m2)
        comb_ref[...] = jnp.where(idx == a1, w1, 0.0) + jnp.where(idx == a2, 1.0 - w1, 0.0)

    gu = jnp.dot(x, w1_ref[0], preferred_element_type=jnp.float32) + b1_ref[0]
    gate = jnp.minimum(gu[:, :_F], _LIMIT)
    up = jnp.clip(gu[:, _F:], -_LIMIT, _LIMIT)
    act = (up + 1.0) * (gate * jax.nn.sigmoid(_ALPHA * gate))
    out = jnp.dot(act, w2_ref[0], preferred_element_type=jnp.float32) + b2_ref[0]
    c = jnp.sum(jnp.where(idx == e, comb_ref[...], 0.0), axis=1, keepdims=True)
    contrib = c * out

    @pl.when(e == 0)
    def _():
        out_ref[...] = contrib

    @pl.when(e > 0)
    def _():
        out_ref[...] += contrib


@jax.jit
def kernel(x_TD, kernel_DE, bias_E, mlp1_weight_EDF2, mlp1_bias_EF2, mlp2_weight_EFD, mlp2_bias_ED):
    x = x_TD.astype(jnp.float32)
    out = pl.pallas_call(
        _moe_body,
        grid=(_E,),
        in_specs=[
            pl.BlockSpec((_T, _D), lambda e: (0, 0)),
            pl.BlockSpec((_D, _E), lambda e: (0, 0)),
            pl.BlockSpec((1, _E), lambda e: (0, 0)),
            pl.BlockSpec((1, _D, 2 * _F), lambda e: (e, 0, 0)),
            pl.BlockSpec((1, 1, 2 * _F), lambda e: (e, 0, 0)),
            pl.BlockSpec((1, _F, _D), lambda e: (e, 0, 0)),
            pl.BlockSpec((1, 1, _D), lambda e: (e, 0, 0)),
        ],
        out_specs=pl.BlockSpec((_T, _D), lambda e: (0, 0)),
        out_shape=jax.ShapeDtypeStruct((_T, _D), jnp.float32),
        scratch_shapes=[pltpu.VMEM((_T, _E), jnp.float32)],
        compiler_params=pltpu.CompilerParams(
            dimension_semantics=("arbitrary",),
        ),
    )(x, kernel_DE, bias_E.reshape(1, _E), mlp1_weight_EDF2,
      mlp1_bias_EF2.reshape(_E, 1, 2 * _F), mlp2_weight_EFD,
      mlp2_bias_ED.reshape(_E, 1, _D))
    return out.astype(jnp.float32)
